# Initial kernel scaffold; baseline (speedup 1.0000x reference)
import jax
import jax.numpy as jnp
from jax.experimental import pallas as pl


def kernel(x, edge_index, W1, b1, W2, b2, W3, b3, Wc1, bc1, Wc2, bc2, Wc3, bc3):
    raise NotImplementedError("write your pallas kernel here")


# speedup vs baseline: 3.9222x; 3.9222x over previous
import functools

import jax
import jax.numpy as jnp
from jax import lax
from jax.experimental import pallas as pl
from jax.experimental.pallas import tpu as pltpu
from jax.experimental.pallas import tpu_sc as plsc

N = 10000
NP = 10240
E = 320000
D = 128
OUT = 10

NC = 2
NS = 16
NW = NC * NS
CH = 128
S = -(-E // (NW * CH))
EP = NW * S * CH
ZR = NP // NS

BLK = 1024
NBLK = NP // BLK

_MESH = plsc.VectorSubcoreMesh(core_axis_name="c", subcore_axis_name="s")
_F32 = jnp.float32
_HIGH = jax.lax.Precision.HIGHEST


@functools.partial(
    pl.kernel,
    out_type=jax.ShapeDtypeStruct((NC, 2, NP), _F32),
    mesh=_MESH,
    scratch_types=[
        pltpu.VMEM((S, CH), jnp.int32),
        pltpu.VMEM((S, CH), jnp.int32),
        pltpu.VMEM((CH,), _F32),
        pltpu.VMEM((ZR,), _F32),
        pltpu.VMEM_SHARED((NP,), _F32),
        pltpu.VMEM_SHARED((NP,), _F32),
    ],
)
def _sc_degrees(src_hbm, dst_hbm, out_hbm, sidx, didx, ones_v, zer_v,
                asrc, adst):
    core = lax.axis_index("c")
    sub = lax.axis_index("s")
    w = core * NS + sub

    @pl.loop(0, CH, step=16)
    def _(i):
        ones_v[pl.ds(i, 16)] = jnp.ones((16,), _F32)

    @pl.loop(0, ZR, step=16)
    def _(i):
        zer_v[pl.ds(i, 16)] = jnp.zeros((16,), _F32)

    pltpu.sync_copy(zer_v, asrc.at[pl.ds(sub * ZR, ZR)])
    pltpu.sync_copy(zer_v, adst.at[pl.ds(sub * ZR, ZR)])
    plsc.subcore_barrier()

    pltpu.sync_copy(src_hbm.at[w], sidx)
    pltpu.sync_copy(dst_hbm.at[w], didx)

    @pl.loop(0, S)
    def _(s):
        pltpu.sync_copy(ones_v, asrc.at[sidx.at[s]], add=True)
        pltpu.sync_copy(ones_v, adst.at[didx.at[s]], add=True)

    plsc.subcore_barrier()
    pltpu.sync_copy(asrc.at[pl.ds(sub * ZR, ZR)],
                    out_hbm.at[core, 0, pl.ds(sub * ZR, ZR)])
    pltpu.sync_copy(adst.at[pl.ds(sub * ZR, ZR)],
                    out_hbm.at[core, 1, pl.ds(sub * ZR, ZR)])


@functools.partial(
    pl.kernel,
    out_type=jax.ShapeDtypeStruct((NC, NP, D), _F32),
    mesh=_MESH,
    scratch_types=[
        pltpu.VMEM((S, CH), jnp.int32),
        pltpu.VMEM((S, CH), jnp.int32),
        pltpu.VMEM((CH, D), _F32),
        pltpu.VMEM((64, D), _F32),
        pltpu.VMEM_SHARED((NP, D), _F32),
    ],
)
def _sc_aggregate(hw_hbm, src_hbm, dst_hbm, out_hbm, sidx, didx, rows,
                  zrows, acc):
    core = lax.axis_index("c")
    sub = lax.axis_index("s")
    w = core * NS + sub

    @pl.loop(0, 64)
    def _(r):
        @pl.loop(0, D, step=16)
        def _(cix):
            zrows[r, pl.ds(cix, 16)] = jnp.zeros((16,), _F32)

    @pl.loop(0, ZR // 64)
    def _(k):
        pltpu.sync_copy(zrows, acc.at[pl.ds(sub * ZR + k * 64, 64)])

    plsc.subcore_barrier()

    pltpu.sync_copy(src_hbm.at[w], sidx)
    pltpu.sync_copy(dst_hbm.at[w], didx)

    @pl.loop(0, S)
    def _(s):
        pltpu.sync_copy(hw_hbm.at[sidx.at[s]], rows)
        pltpu.sync_copy(rows, acc.at[didx.at[s]], add=True)

    plsc.subcore_barrier()
    pltpu.sync_copy(acc.at[pl.ds(sub * ZR, ZR)],
                    out_hbm.at[core, pl.ds(sub * ZR, ZR)])


def _prep_body(cnt_ref, x_ref, w1_ref, hw_ref, dout_ref, sd_ref, din_ref):
    cs = cnt_ref[0, 0] + cnt_ref[1, 0]
    cd = cnt_ref[0, 1] + cnt_ref[1, 1]
    dout = lax.rsqrt(jnp.maximum(cs, 1.0))
    din = lax.rsqrt(jnp.maximum(cd, 1.0))
    dout_ref[...] = dout
    din_ref[...] = din
    sd_ref[...] = dout * din
    xs = x_ref[...] * dout
    hw_ref[...] = jnp.dot(xs, w1_ref[...], precision=_HIGH,
                          preferred_element_type=_F32)


def _layer_body(p_ref, sd_ref, dout_ref, b_ref, w_ref, out_ref):
    agg = p_ref[0] + p_ref[1]
    h = jnp.maximum(agg * sd_ref[...] + b_ref[...] * dout_ref[...], 0.0)
    out_ref[...] = jnp.dot(h, w_ref[...], precision=_HIGH,
                           preferred_element_type=_F32)


def _final_body(p_ref, din_ref, b3_ref, wc1_ref, bc1_ref, wc2_ref, bc2_ref,
                wc3_ref, bc3_ref, out_ref, acc_ref):
    i = pl.program_id(0)

    @pl.when(i == 0)
    def _():
        acc_ref[...] = jnp.zeros_like(acc_ref)

    agg = p_ref[0] + p_ref[1]
    h = jnp.maximum(agg * din_ref[...] + b3_ref[...], 0.0)
    row = lax.broadcasted_iota(jnp.int32, (BLK, 1), 0) + i * BLK
    h = jnp.where(row < N, h, 0.0)
    acc_ref[...] = jnp.maximum(acc_ref[...], jnp.max(h, axis=0, keepdims=True))

    @pl.when(i == NBLK - 1)
    def _():
        hg = acc_ref[...]
        z = jnp.dot(hg, wc1_ref[...], precision=_HIGH,
                    preferred_element_type=_F32) + bc1_ref[...]
        z = jnp.maximum(z, 0.0)
        z = jnp.dot(z, wc2_ref[...], precision=_HIGH,
                    preferred_element_type=_F32) + bc2_ref[...]
        z = jnp.maximum(z, 0.0)
        out_ref[...] = jnp.dot(z, wc3_ref[...], precision=_HIGH,
                               preferred_element_type=_F32) + bc3_ref[...]


def _row_specs(*shapes):
    specs = []
    for shp in shapes:
        if shp == "p":
            specs.append(pl.BlockSpec((NC, BLK, D), lambda i: (0, i, 0)))
        elif shp == "col":
            specs.append(pl.BlockSpec((BLK, 1), lambda i: (i, 0)))
        elif shp == "cnt":
            specs.append(pl.BlockSpec((NC, 2, BLK, 1), lambda i: (0, 0, i, 0)))
        elif shp == "x":
            specs.append(pl.BlockSpec((BLK, D), lambda i: (i, 0)))
        else:
            specs.append(
                pl.BlockSpec(shp, lambda i, r=len(shp): (0,) * r))
    return specs


def _tc_prep(cnt4, xp, W1):
    return pl.pallas_call(
        _prep_body,
        grid=(NBLK,),
        in_specs=_row_specs("cnt", "x", (D, D)),
        out_specs=[
            pl.BlockSpec((BLK, D), lambda i: (i, 0)),
            pl.BlockSpec((BLK, 1), lambda i: (i, 0)),
            pl.BlockSpec((BLK, 1), lambda i: (i, 0)),
            pl.BlockSpec((BLK, 1), lambda i: (i, 0)),
        ],
        out_shape=[
            jax.ShapeDtypeStruct((NP, D), _F32),
            jax.ShapeDtypeStruct((NP, 1), _F32),
            jax.ShapeDtypeStruct((NP, 1), _F32),
            jax.ShapeDtypeStruct((NP, 1), _F32),
        ],
    )(cnt4, xp, W1)


def _tc_layer(p, sd, dout, b, W):
    return pl.pallas_call(
        _layer_body,
        grid=(NBLK,),
        in_specs=_row_specs("p", "col", "col", (1, D), (D, D)),
        out_specs=pl.BlockSpec((BLK, D), lambda i: (i, 0)),
        out_shape=jax.ShapeDtypeStruct((NP, D), _F32),
    )(p, sd, dout, b, W)


def _tc_final(p, din, b3, Wc1, bc1, Wc2p, bc2p, Wc3p, bc3p):
    return pl.pallas_call(
        _final_body,
        grid=(NBLK,),
        in_specs=_row_specs("p", "col", (1, D), (D, D), (1, D), (D, D),
                            (1, D), (D, D), (1, D)),
        out_specs=pl.BlockSpec((1, D), lambda i: (0, 0)),
        out_shape=jax.ShapeDtypeStruct((1, D), _F32),
        scratch_shapes=[pltpu.VMEM((1, D), _F32)],
    )(p, din, b3, Wc1, bc1, Wc2p, bc2p, Wc3p, bc3p)


def kernel(x, edge_index, W1, b1, W2, b2, W3, b3, Wc1, bc1, Wc2, bc2,
           Wc3, bc3):
    src = edge_index[0]
    dst = edge_index[1]
    pad = jnp.full((EP - E,), N, jnp.int32)
    srcp = jnp.concatenate([src, pad]).reshape(NW, S, CH)
    dstp = jnp.concatenate([dst, pad]).reshape(NW, S, CH)
    xp = jnp.pad(x, ((0, NP - N), (0, 0)))

    cnt = _sc_degrees(srcp, dstp)
    cnt4 = cnt.reshape(NC, 2, NP, 1)

    hw1, dout, sd, din = _tc_prep(cnt4, xp, W1)
    p1 = _sc_aggregate(hw1, srcp, dstp)
    hw2 = _tc_layer(p1, sd, dout, b1.reshape(1, D), W2)
    p2 = _sc_aggregate(hw2, srcp, dstp)
    hw3 = _tc_layer(p2, sd, dout, b2.reshape(1, D), W3)
    p3 = _sc_aggregate(hw3, srcp, dstp)

    Wc2p = jnp.pad(Wc2, ((0, 0), (0, D - Wc2.shape[1])))
    bc2p = jnp.pad(bc2, (0, D - bc2.shape[0])).reshape(1, D)
    Wc3p = jnp.pad(Wc3, ((0, D - Wc3.shape[0]), (0, D - Wc3.shape[1])))
    bc3p = jnp.pad(bc3, (0, D - bc3.shape[0])).reshape(1, D)

    logits = _tc_final(p3, din, b3.reshape(1, D), Wc1,
                       bc1.reshape(1, D), Wc2p, bc2p, Wc3p, bc3p)
    return logits[:, :OUT]
